# baseline (speedup 1.0000x reference)
import functools

import jax
import jax.numpy as jnp
from jax import lax
from jax.experimental import pallas as pl
from jax.experimental.pallas import tpu as pltpu
from jax.experimental.pallas import tpu_sc as plsc

NC, NS = 2, 16
NW = NC * NS
CHUNK = 128
LANES = 16
MASK_HI = 0xFFFF0000

INV_2PI = 0.15915493667125702
RND_MAGIC = 12582912.0
S0 = 6.283055994859666
S1 = -41.331226406885634
S2 = 81.36701207816412
S3 = -74.47917011197654
S4 = 32.78367310635748


def _sc_gather_pack(word_tab, aux_tab, widx, aidx, BL, H):
  per_w = BL // NW
  n_chunks = per_w // CHUNK
  n_pairs = n_chunks // 2
  mesh = plsc.VectorSubcoreMesh(core_axis_name="c", subcore_axis_name="s",
                                num_cores=NC, num_subcores=NS)

  @functools.partial(
      pl.kernel,
      out_type=jax.ShapeDtypeStruct((BL, H), jnp.uint32),
      mesh=mesh,
      scratch_types=[
          pltpu.VMEM((per_w,), jnp.int32),
          pltpu.VMEM((per_w,), jnp.int32),
          pltpu.VMEM((CHUNK, H), jnp.uint32),
          pltpu.VMEM((CHUNK, H), jnp.uint32),
          pltpu.VMEM((CHUNK, H), jnp.uint32),
          pltpu.VMEM((CHUNK, H), jnp.uint32),
          pltpu.VMEM((CHUNK, H), jnp.uint32),
          pltpu.SemaphoreType.DMA,
          pltpu.SemaphoreType.DMA,
          pltpu.SemaphoreType.DMA,
          pltpu.SemaphoreType.DMA,
      ],
  )
  def k(word_hbm, aux_hbm, widx_hbm, aidx_hbm, out_hbm,
        widx_v, aidx_v, wbuf0, abuf0, wbuf1, abuf1, obuf,
        sem_w0, sem_a0, sem_w1, sem_a1):
    wid = lax.axis_index("s") * NC + lax.axis_index("c")
    base = pl.multiple_of(wid * per_w, per_w)
    pltpu.sync_copy(widx_hbm.at[pl.ds(base, per_w)], widx_v)
    pltpu.sync_copy(aidx_hbm.at[pl.ds(base, per_w)], aidx_v)

    def gather(c, wbuf, abuf, sem_w, sem_a):
      off = pl.multiple_of(c * CHUNK, CHUNK)
      pltpu.async_copy(word_hbm.at[widx_v.at[pl.ds(off, CHUNK)]], wbuf, sem_w)
      pltpu.async_copy(aux_hbm.at[aidx_v.at[pl.ds(off, CHUNK)]], abuf, sem_a)

    def wait(wbuf, abuf, sem_w, sem_a):
      pltpu.make_async_copy(word_hbm.at[widx_v.at[pl.ds(0, CHUNK)]],
                            wbuf, sem_w).wait()
      pltpu.make_async_copy(aux_hbm.at[aidx_v.at[pl.ds(0, CHUNK)]],
                            abuf, sem_a).wait()

    def pack_store(c, wbuf, abuf):
      mask = jnp.uint32(MASK_HI)
      def pack_row(t, carry):
        for kk in range(H // LANES):
          w16 = wbuf[t, pl.ds(LANES * kk, LANES)]
          a16 = abuf[t, pl.ds(LANES * kk, LANES)]
          obuf[t, pl.ds(LANES * kk, LANES)] = (w16 >> 16) | (a16 & mask)
        return carry
      lax.fori_loop(0, CHUNK, pack_row, 0)
      dst = pl.multiple_of(base + c * CHUNK, CHUNK)
      pltpu.sync_copy(obuf, out_hbm.at[pl.ds(dst, CHUNK)])

    gather(0, wbuf0, abuf0, sem_w0, sem_a0)

    def body(i, carry):
      c0 = 2 * i
      gather(c0 + 1, wbuf1, abuf1, sem_w1, sem_a1)
      wait(wbuf0, abuf0, sem_w0, sem_a0)
      pack_store(c0, wbuf0, abuf0)

      @pl.when(i < n_pairs - 1)
      def _():
        gather(c0 + 2, wbuf0, abuf0, sem_w0, sem_a0)

      wait(wbuf1, abuf1, sem_w1, sem_a1)
      pack_store(c0 + 1, wbuf1, abuf1)
      return carry

    lax.fori_loop(0, n_pairs, body, 0)

  return k(word_tab, aux_tab, widx, aidx)


def _fast_sin(x):
  y = x * INV_2PI
  k = (y + RND_MAGIC) - RND_MAGIC
  t = y - k
  u = t * t
  return t * (S0 + u * (S1 + u * (S2 + u * (S3 + u * S4))))


def _tc_math(packed, deltas, ages, wc, wta, cw, cphi,
             b, gamma, beta, BL, H, T, TB=1024):
  nb = BL // TB

  def body(p_ref, d_ref, ag_ref, wc_ref, wta_ref,
           cw_ref, cphi_ref, b_ref, gm_ref, bt_ref, out_ref):
    cwv = cw_ref[...]
    cph = cphi_ref[...]
    ph = jnp.concatenate(
        [d_ref[...] * cwv[:, :T] + cph[:, :T],
         ag_ref[...] * cwv[:, T:] + cph[:, T:]], axis=1)
    feats = _fast_sin(ph)
    p = p_ref[...]
    code = lax.bitcast_convert_type(p << 16, jnp.float32)
    aux_f = lax.bitcast_convert_type(p & jnp.uint32(MASK_HI), jnp.float32)
    acc = jnp.dot(code, wc_ref[...], preferred_element_type=jnp.float32)
    acc += jnp.dot(feats, wta_ref[...], preferred_element_type=jnp.float32)
    tok = jnp.tanh(acc + b_ref[...]) + aux_f
    mu = jnp.mean(tok, axis=1, keepdims=True)
    var = jnp.mean(jnp.square(tok - mu), axis=1, keepdims=True)
    out_ref[...] = ((tok - mu) * lax.rsqrt(var + 1e-12)
                    * gm_ref[...] + bt_ref[...])

  full = lambda r, c: pl.BlockSpec((r, c), lambda i: (0, 0))
  return pl.pallas_call(
      body,
      grid=(nb,),
      in_specs=[
          pl.BlockSpec((TB, H), lambda i: (i, 0)),
          pl.BlockSpec((TB, 1), lambda i: (i, 0)),
          pl.BlockSpec((TB, 1), lambda i: (i, 0)),
          full(H, H), full(2 * T, H),
          full(1, 2 * T), full(1, 2 * T),
          full(1, H), full(1, H), full(1, H),
      ],
      out_specs=pl.BlockSpec((TB, H), lambda i: (i, 0)),
      out_shape=jax.ShapeDtypeStruct((BL, H), jnp.float32),
      compiler_params=pltpu.CompilerParams(
          dimension_semantics=("arbitrary",)),
  )(packed, deltas, ages, wc, wta, cw, cphi, b, gamma, beta)


def kernel(input_ids, token_type_ids, time_stamps, ages, visit_orders,
           visit_segments, word_emb, type_emb, order_emb, seg_emb,
           time_w, time_phi, age_w, age_phi, proj_W, proj_b,
           ln_gamma, ln_beta):
  B, Lx = input_ids.shape
  V, H = word_emb.shape
  T = time_w.shape[1]
  n_type, n_seg, n_order = type_emb.shape[0], seg_emb.shape[0], order_emb.shape[0]
  BL = B * Lx

  aux_tab = ((type_emb[:, None, :] + seg_emb[None, :, :])
             .reshape(n_type * n_seg, H)[:, None, :]
             + order_emb[None, :, :]).reshape(n_type * n_seg * n_order, H)
  aidx = ((token_type_ids * n_seg + visit_segments) * n_order
          + visit_orders).reshape(BL).astype(jnp.int32)
  widx = input_ids.reshape(BL).astype(jnp.int32)

  packed = _sc_gather_pack(lax.bitcast_convert_type(word_emb, jnp.uint32),
                           lax.bitcast_convert_type(aux_tab, jnp.uint32),
                           widx, aidx, BL, H)

  deltas = jnp.concatenate(
      [time_stamps[:, :1] * 0.0, time_stamps[:, 1:] - time_stamps[:, :-1]],
      axis=-1).reshape(BL, 1)
  ages2 = ages.reshape(BL, 1)

  out = _tc_math(packed, deltas, ages2,
                 proj_W[:H],
                 proj_W[H:],
                 jnp.concatenate([time_w, age_w], axis=1),
                 jnp.concatenate([time_phi, age_phi], axis=1),
                 proj_b.reshape(1, H), ln_gamma.reshape(1, H),
                 ln_beta.reshape(1, H), BL, H, T)
  return out.reshape(B, Lx, H)


# speedup vs baseline: 1.5795x; 1.5795x over previous
import functools

import jax
import jax.numpy as jnp
from jax import lax
from jax.experimental import pallas as pl
from jax.experimental.pallas import tpu as pltpu
from jax.experimental.pallas import tpu_sc as plsc

NC, NS = 2, 16
NW = NC * NS
CHUNK = 128
LANES = 16
MASK_HI = 0xFFFF0000

INV_2PI = 0.15915493667125702
RND_MAGIC = 12582912.0
S0 = 6.283055994859666
S1 = -41.331226406885634
S2 = 81.36701207816412
S3 = -74.47917011197654
S4 = 32.78367310635748


def _sc_gather_pack(word_tab, aux_tab, widx, aidx, BL, H):
  per_w = BL // NW
  n_chunks = per_w // CHUNK
  n_pairs = n_chunks // 2
  mesh = plsc.VectorSubcoreMesh(core_axis_name="c", subcore_axis_name="s",
                                num_cores=NC, num_subcores=NS)

  @functools.partial(
      pl.kernel,
      out_type=jax.ShapeDtypeStruct((BL, H), jnp.uint32),
      mesh=mesh,
      scratch_types=[
          pltpu.VMEM((per_w,), jnp.int32),
          pltpu.VMEM((per_w,), jnp.int32),
          pltpu.VMEM((CHUNK, H), jnp.uint32),
          pltpu.VMEM((CHUNK, H), jnp.uint32),
          pltpu.VMEM((CHUNK, H), jnp.uint32),
          pltpu.VMEM((CHUNK, H), jnp.uint32),
          pltpu.VMEM((CHUNK, H), jnp.uint32),
          pltpu.SemaphoreType.DMA,
          pltpu.SemaphoreType.DMA,
          pltpu.SemaphoreType.DMA,
          pltpu.SemaphoreType.DMA,
      ],
  )
  def k(word_hbm, aux_hbm, widx_hbm, aidx_hbm, out_hbm,
        widx_v, aidx_v, wbuf0, abuf0, wbuf1, abuf1, obuf,
        sem_w0, sem_a0, sem_w1, sem_a1):
    wid = lax.axis_index("s") * NC + lax.axis_index("c")
    base = pl.multiple_of(wid * per_w, per_w)
    pltpu.sync_copy(widx_hbm.at[pl.ds(base, per_w)], widx_v)
    pltpu.sync_copy(aidx_hbm.at[pl.ds(base, per_w)], aidx_v)

    def gather(c, wbuf, abuf, sem_w, sem_a):
      off = pl.multiple_of(c * CHUNK, CHUNK)
      pltpu.async_copy(word_hbm.at[widx_v.at[pl.ds(off, CHUNK)]], wbuf, sem_w)
      pltpu.async_copy(aux_hbm.at[aidx_v.at[pl.ds(off, CHUNK)]], abuf, sem_a)

    def wait(wbuf, abuf, sem_w, sem_a):
      pltpu.make_async_copy(word_hbm.at[widx_v.at[pl.ds(0, CHUNK)]],
                            wbuf, sem_w).wait()
      pltpu.make_async_copy(aux_hbm.at[aidx_v.at[pl.ds(0, CHUNK)]],
                            abuf, sem_a).wait()

    def pack_store(c, wbuf, abuf):
      mask = jnp.uint32(MASK_HI)
      def pack_row(t, carry):
        for kk in range(H // LANES):
          w16 = wbuf[t, pl.ds(LANES * kk, LANES)]
          a16 = abuf[t, pl.ds(LANES * kk, LANES)]
          obuf[t, pl.ds(LANES * kk, LANES)] = (w16 >> 16) | (a16 & mask)
        return carry
      lax.fori_loop(0, CHUNK, pack_row, 0)
      dst = pl.multiple_of(base + c * CHUNK, CHUNK)
      pltpu.sync_copy(obuf, out_hbm.at[pl.ds(dst, CHUNK)])

    gather(0, wbuf0, abuf0, sem_w0, sem_a0)

    def body(i, carry):
      c0 = 2 * i
      gather(c0 + 1, wbuf1, abuf1, sem_w1, sem_a1)
      wait(wbuf0, abuf0, sem_w0, sem_a0)
      pack_store(c0, wbuf0, abuf0)

      @pl.when(i < n_pairs - 1)
      def _():
        gather(c0 + 2, wbuf0, abuf0, sem_w0, sem_a0)

      wait(wbuf1, abuf1, sem_w1, sem_a1)
      pack_store(c0 + 1, wbuf1, abuf1)
      return carry

    lax.fori_loop(0, n_pairs, body, 0)

  return k(word_tab, aux_tab, widx, aidx)


def _fast_sin(x):
  y = x * INV_2PI
  k = (y + RND_MAGIC) - RND_MAGIC
  t = y - k
  u = t * t
  return t * (S0 + u * (S1 + u * (S2 + u * (S3 + u * S4))))


def _tc_math(packed, deltas, ages, wc, wta, cwc, cphc,
             b, gamma, beta, BL, H, T, TB=2048):
  nb = BL // TB

  def body(p_ref, d_ref, ag_ref, wc_ref, wta_ref,
           cw_ref, cphi_ref, b_ref, gm_ref, bt_ref, out_ref):
    d = d_ref[0]
    a = ag_ref[0]
    cwv = cw_ref[...]
    cph = cphi_ref[...]
    ph = jnp.concatenate(
        [d * cwv[:T] + cph[:T],
         a * cwv[T:] + cph[T:]], axis=0)
    feats_t = _fast_sin(ph)
    p = p_ref[...]
    code = lax.bitcast_convert_type(p << 16, jnp.float32)
    aux_f = lax.bitcast_convert_type(p & jnp.uint32(MASK_HI), jnp.float32)
    acc = jnp.dot(code, wc_ref[...], preferred_element_type=jnp.float32)
    acc += lax.dot_general(feats_t, wta_ref[...], (((0,), (0,)), ((), ())),
                           preferred_element_type=jnp.float32)
    tok = jnp.tanh(acc + b_ref[...]) + aux_f
    mu = jnp.mean(tok, axis=1, keepdims=True)
    var = jnp.mean(jnp.square(tok - mu), axis=1, keepdims=True)
    out_ref[...] = ((tok - mu) * lax.rsqrt(var + 1e-12)
                    * gm_ref[...] + bt_ref[...])

  full = lambda r, c: pl.BlockSpec((r, c), lambda i: (0, 0))
  return pl.pallas_call(
      body,
      grid=(nb,),
      in_specs=[
          pl.BlockSpec((TB, H), lambda i: (i, 0)),
          pl.BlockSpec((1, 1, TB), lambda i: (i, 0, 0)),
          pl.BlockSpec((1, 1, TB), lambda i: (i, 0, 0)),
          full(H, H), full(2 * T, H),
          full(2 * T, 1), full(2 * T, 1),
          full(1, H), full(1, H), full(1, H),
      ],
      out_specs=pl.BlockSpec((TB, H), lambda i: (i, 0)),
      out_shape=jax.ShapeDtypeStruct((BL, H), jnp.float32),
      compiler_params=pltpu.CompilerParams(
          dimension_semantics=("arbitrary",)),
  )(packed, deltas, ages, wc, wta, cwc, cphc, b, gamma, beta)


def kernel(input_ids, token_type_ids, time_stamps, ages, visit_orders,
           visit_segments, word_emb, type_emb, order_emb, seg_emb,
           time_w, time_phi, age_w, age_phi, proj_W, proj_b,
           ln_gamma, ln_beta):
  B, Lx = input_ids.shape
  V, H = word_emb.shape
  T = time_w.shape[1]
  n_type, n_seg, n_order = type_emb.shape[0], seg_emb.shape[0], order_emb.shape[0]
  BL = B * Lx

  aux_tab = ((type_emb[:, None, :] + seg_emb[None, :, :])
             .reshape(n_type * n_seg, H)[:, None, :]
             + order_emb[None, :, :]).reshape(n_type * n_seg * n_order, H)
  aidx = ((token_type_ids * n_seg + visit_segments) * n_order
          + visit_orders).reshape(BL).astype(jnp.int32)
  widx = input_ids.reshape(BL).astype(jnp.int32)

  packed = _sc_gather_pack(lax.bitcast_convert_type(word_emb, jnp.uint32),
                           lax.bitcast_convert_type(aux_tab, jnp.uint32),
                           widx, aidx, BL, H)

  TB = 2048
  deltas = jnp.concatenate(
      [time_stamps[:, :1] * 0.0, time_stamps[:, 1:] - time_stamps[:, :-1]],
      axis=-1).reshape(BL // TB, 1, TB)
  ages2 = ages.reshape(BL // TB, 1, TB)

  out = _tc_math(packed, deltas, ages2,
                 proj_W[:H],
                 proj_W[H:],
                 jnp.concatenate([time_w, age_w], axis=1).reshape(2 * T, 1),
                 jnp.concatenate([time_phi, age_phi], axis=1).reshape(2 * T, 1),
                 proj_b.reshape(1, H), ln_gamma.reshape(1, H),
                 ln_beta.reshape(1, H), BL, H, T, TB=TB)
  return out.reshape(B, Lx, H)


# speedup vs baseline: 1.7581x; 1.1130x over previous
import functools

import jax
import jax.numpy as jnp
from jax import lax
from jax.experimental import pallas as pl
from jax.experimental.pallas import tpu as pltpu
from jax.experimental.pallas import tpu_sc as plsc

NC, NS = 2, 16
NW = NC * NS
CHUNK = 128
LANES = 16
MASK_HI = 0xFFFF0000

INV_2PI = 0.15915493667125702
RND_MAGIC = 12582912.0
S0 = 6.283055994859666
S1 = -41.331226406885634
S2 = 81.36701207816412
S3 = -74.47917011197654
S4 = 32.78367310635748


def _sc_gather_pack(word_tab, aux_tab, widx, aidx, BL, H):
  per_w = BL // NW
  n_chunks = per_w // CHUNK
  n_pairs = n_chunks // 2
  tail = n_chunks % 2
  mesh = plsc.VectorSubcoreMesh(core_axis_name="c", subcore_axis_name="s",
                                num_cores=NC, num_subcores=NS)

  @functools.partial(
      pl.kernel,
      out_type=jax.ShapeDtypeStruct((BL, H), jnp.uint32),
      mesh=mesh,
      scratch_types=[
          pltpu.VMEM((per_w,), jnp.int32),
          pltpu.VMEM((per_w,), jnp.int32),
          pltpu.VMEM((CHUNK, H), jnp.uint32),
          pltpu.VMEM((CHUNK, H), jnp.uint32),
          pltpu.VMEM((CHUNK, H), jnp.uint32),
          pltpu.VMEM((CHUNK, H), jnp.uint32),
          pltpu.VMEM((CHUNK, H), jnp.uint32),
          pltpu.SemaphoreType.DMA,
          pltpu.SemaphoreType.DMA,
          pltpu.SemaphoreType.DMA,
          pltpu.SemaphoreType.DMA,
      ],
  )
  def k(word_hbm, aux_hbm, widx_hbm, aidx_hbm, out_hbm,
        widx_v, aidx_v, wbuf0, abuf0, wbuf1, abuf1, obuf,
        sem_w0, sem_a0, sem_w1, sem_a1):
    wid = lax.axis_index("s") * NC + lax.axis_index("c")
    base = pl.multiple_of(wid * per_w, per_w)
    pltpu.sync_copy(widx_hbm.at[pl.ds(base, per_w)], widx_v)
    pltpu.sync_copy(aidx_hbm.at[pl.ds(base, per_w)], aidx_v)

    def gather(c, wbuf, abuf, sem_w, sem_a):
      off = pl.multiple_of(c * CHUNK, CHUNK)
      pltpu.async_copy(word_hbm.at[widx_v.at[pl.ds(off, CHUNK)]], wbuf, sem_w)
      pltpu.async_copy(aux_hbm.at[aidx_v.at[pl.ds(off, CHUNK)]], abuf, sem_a)

    def wait(wbuf, abuf, sem_w, sem_a):
      pltpu.make_async_copy(word_hbm.at[widx_v.at[pl.ds(0, CHUNK)]],
                            wbuf, sem_w).wait()
      pltpu.make_async_copy(aux_hbm.at[aidx_v.at[pl.ds(0, CHUNK)]],
                            abuf, sem_a).wait()

    def pack_store(c, wbuf, abuf):
      mask = jnp.uint32(MASK_HI)
      def pack_row(t, carry):
        for kk in range(H // LANES):
          w16 = wbuf[t, pl.ds(LANES * kk, LANES)]
          a16 = abuf[t, pl.ds(LANES * kk, LANES)]
          obuf[t, pl.ds(LANES * kk, LANES)] = (w16 >> 16) | (a16 & mask)
        return carry
      lax.fori_loop(0, CHUNK, pack_row, 0)
      dst = pl.multiple_of(base + c * CHUNK, CHUNK)
      pltpu.sync_copy(obuf, out_hbm.at[pl.ds(dst, CHUNK)])

    gather(0, wbuf0, abuf0, sem_w0, sem_a0)

    def body(i, carry):
      c0 = 2 * i
      gather(c0 + 1, wbuf1, abuf1, sem_w1, sem_a1)
      wait(wbuf0, abuf0, sem_w0, sem_a0)
      pack_store(c0, wbuf0, abuf0)

      @pl.when(c0 + 2 < n_chunks)
      def _():
        gather(c0 + 2, wbuf0, abuf0, sem_w0, sem_a0)

      wait(wbuf1, abuf1, sem_w1, sem_a1)
      pack_store(c0 + 1, wbuf1, abuf1)
      return carry

    lax.fori_loop(0, n_pairs, body, 0)
    if tail:
      wait(wbuf0, abuf0, sem_w0, sem_a0)
      pack_store(n_chunks - 1, wbuf0, abuf0)

  return k(word_tab, aux_tab, widx, aidx)


def _fast_sin(x):
  y = x * INV_2PI
  k = (y + RND_MAGIC) - RND_MAGIC
  t = y - k
  u = t * t
  return t * (S0 + u * (S1 + u * (S2 + u * (S3 + u * S4))))


def _tc_math(prev, packed, deltas, ages, wc, wta, cwc, cphc,
             b, gamma, beta, BL, H, T, TB, off):
  nb = packed.shape[0] // TB

  def body(*refs):
    if prev is None:
      (p_ref, d_ref, ag_ref, wc_ref, wta_ref,
       cw_ref, cphi_ref, b_ref, gm_ref, bt_ref, out_ref) = refs
    else:
      (_, p_ref, d_ref, ag_ref, wc_ref, wta_ref,
       cw_ref, cphi_ref, b_ref, gm_ref, bt_ref, out_ref) = refs
    d = d_ref[0]
    a = ag_ref[0]
    cwv = cw_ref[...]
    cph = cphi_ref[...]
    ph = jnp.concatenate(
        [d * cwv[:T] + cph[:T],
         a * cwv[T:] + cph[T:]], axis=0)
    feats_t = _fast_sin(ph)
    p = p_ref[...]
    code = lax.bitcast_convert_type(p << 16, jnp.float32)
    aux_f = lax.bitcast_convert_type(p & jnp.uint32(MASK_HI), jnp.float32)
    acc = jnp.dot(code, wc_ref[...], preferred_element_type=jnp.float32)
    acc += lax.dot_general(feats_t, wta_ref[...], (((0,), (0,)), ((), ())),
                           preferred_element_type=jnp.float32)
    tok = jnp.tanh(acc + b_ref[...]) + aux_f
    mu = jnp.mean(tok, axis=1, keepdims=True)
    var = jnp.mean(jnp.square(tok - mu), axis=1, keepdims=True)
    out_ref[...] = ((tok - mu) * lax.rsqrt(var + 1e-12)
                    * gm_ref[...] + bt_ref[...])

  full = lambda r, c: pl.BlockSpec((r, c), lambda i: (0, 0))
  in_specs = [
      pl.BlockSpec((TB, H), lambda i: (i, 0)),
      pl.BlockSpec((1, 1, TB), lambda i: (i, 0, 0)),
      pl.BlockSpec((1, 1, TB), lambda i: (i, 0, 0)),
      full(H, H), full(2 * T, H),
      full(2 * T, 1), full(2 * T, 1),
      full(1, H), full(1, H), full(1, H),
  ]
  args = (packed, deltas, ages, wc, wta, cwc, cphc, b, gamma, beta)
  aliases = {}
  if prev is not None:
    in_specs = [pl.BlockSpec(memory_space=pltpu.MemorySpace.HBM)] + in_specs
    args = (prev,) + args
    aliases = {0: 0}
  return pl.pallas_call(
      body,
      grid=(nb,),
      in_specs=in_specs,
      out_specs=pl.BlockSpec((TB, H), lambda i: (i + off, 0)),
      out_shape=jax.ShapeDtypeStruct((BL, H), jnp.float32),
      input_output_aliases=aliases,
      compiler_params=pltpu.CompilerParams(
          dimension_semantics=("arbitrary",)),
  )(*args)


def kernel(input_ids, token_type_ids, time_stamps, ages, visit_orders,
           visit_segments, word_emb, type_emb, order_emb, seg_emb,
           time_w, time_phi, age_w, age_phi, proj_W, proj_b,
           ln_gamma, ln_beta):
  B, Lx = input_ids.shape
  V, H = word_emb.shape
  T = time_w.shape[1]
  n_type, n_seg, n_order = type_emb.shape[0], seg_emb.shape[0], order_emb.shape[0]
  BL = B * Lx

  aux_tab = ((type_emb[:, None, :] + seg_emb[None, :, :])
             .reshape(n_type * n_seg, H)[:, None, :]
             + order_emb[None, :, :]).reshape(n_type * n_seg * n_order, H)
  aidx = ((token_type_ids * n_seg + visit_segments) * n_order
          + visit_orders).reshape(BL).astype(jnp.int32)
  widx = input_ids.reshape(BL).astype(jnp.int32)

  word_u = lax.bitcast_convert_type(word_emb, jnp.uint32)
  aux_u = lax.bitcast_convert_type(aux_tab, jnp.uint32)

  TB = 2048
  NSLICE = 2
  SL = BL // NSLICE
  deltas = jnp.concatenate(
      [time_stamps[:, :1] * 0.0, time_stamps[:, 1:] - time_stamps[:, :-1]],
      axis=-1).reshape(BL // TB, 1, TB)
  ages2 = ages.reshape(BL // TB, 1, TB)
  wc = proj_W[:H]
  wta = proj_W[H:]
  cwc = jnp.concatenate([time_w, age_w], axis=1).reshape(2 * T, 1)
  cphc = jnp.concatenate([time_phi, age_phi], axis=1).reshape(2 * T, 1)
  bb = proj_b.reshape(1, H)
  gm = ln_gamma.reshape(1, H)
  bt = ln_beta.reshape(1, H)

  nbs = SL // TB
  packed = [
      _sc_gather_pack(word_u, aux_u,
                      widx[s * SL:(s + 1) * SL], aidx[s * SL:(s + 1) * SL],
                      SL, H)
      for s in range(NSLICE)
  ]
  out = None
  for s in range(NSLICE):
    out = _tc_math(out, packed[s],
                   deltas[s * nbs:(s + 1) * nbs], ages2[s * nbs:(s + 1) * nbs],
                   wc, wta, cwc, cphc, bb, gm, bt,
                   BL, H, T, TB, off=s * nbs)
  return out.reshape(B, Lx, H)


# speedup vs baseline: 1.8319x; 1.0420x over previous
import functools

import jax
import jax.numpy as jnp
from jax import lax
from jax.experimental import pallas as pl
from jax.experimental.pallas import tpu as pltpu
from jax.experimental.pallas import tpu_sc as plsc

NC, NS = 2, 16
NW = NC * NS
CHUNK = 128
LANES = 16
MASK_HI = 0xFFFF0000

INV_2PI = 0.15915493667125702
RND_MAGIC = 12582912.0
S0 = 6.283055994859666
S1 = -41.331226406885634
S2 = 81.36701207816412
S3 = -74.47917011197654
S4 = 32.78367310635748


def _sc_gather_pack(word_tab, aux_tab, widx, aidx, BL, H):
  per_w = BL // NW
  chunk = next(c for c in (CHUNK, 80, 64, 40, 32, 16, 8) if per_w % c == 0)
  n_chunks = per_w // chunk
  n_pairs = n_chunks // 2
  tail = n_chunks % 2
  mesh = plsc.VectorSubcoreMesh(core_axis_name="c", subcore_axis_name="s",
                                num_cores=NC, num_subcores=NS)

  @functools.partial(
      pl.kernel,
      out_type=jax.ShapeDtypeStruct((BL, H), jnp.uint32),
      mesh=mesh,
      scratch_types=[
          pltpu.VMEM((per_w,), jnp.int32),
          pltpu.VMEM((per_w,), jnp.int32),
          pltpu.VMEM((chunk, H), jnp.uint32),
          pltpu.VMEM((chunk, H), jnp.uint32),
          pltpu.VMEM((chunk, H), jnp.uint32),
          pltpu.VMEM((chunk, H), jnp.uint32),
          pltpu.VMEM((chunk, H), jnp.uint32),
          pltpu.SemaphoreType.DMA,
          pltpu.SemaphoreType.DMA,
          pltpu.SemaphoreType.DMA,
          pltpu.SemaphoreType.DMA,
      ],
  )
  def k(word_hbm, aux_hbm, widx_hbm, aidx_hbm, out_hbm,
        widx_v, aidx_v, wbuf0, abuf0, wbuf1, abuf1, obuf,
        sem_w0, sem_a0, sem_w1, sem_a1):
    wid = lax.axis_index("s") * NC + lax.axis_index("c")
    base = pl.multiple_of(wid * per_w, per_w)
    pltpu.sync_copy(widx_hbm.at[pl.ds(base, per_w)], widx_v)
    pltpu.sync_copy(aidx_hbm.at[pl.ds(base, per_w)], aidx_v)

    def gather(c, wbuf, abuf, sem_w, sem_a):
      off = pl.multiple_of(c * chunk, chunk)
      pltpu.async_copy(word_hbm.at[widx_v.at[pl.ds(off, chunk)]], wbuf, sem_w)
      pltpu.async_copy(aux_hbm.at[aidx_v.at[pl.ds(off, chunk)]], abuf, sem_a)

    def wait(wbuf, abuf, sem_w, sem_a):
      pltpu.make_async_copy(word_hbm.at[widx_v.at[pl.ds(0, chunk)]],
                            wbuf, sem_w).wait()
      pltpu.make_async_copy(aux_hbm.at[aidx_v.at[pl.ds(0, chunk)]],
                            abuf, sem_a).wait()

    def pack_store(c, wbuf, abuf):
      mask = jnp.uint32(MASK_HI)
      def pack_row(t, carry):
        for kk in range(H // LANES):
          w16 = wbuf[t, pl.ds(LANES * kk, LANES)]
          a16 = abuf[t, pl.ds(LANES * kk, LANES)]
          obuf[t, pl.ds(LANES * kk, LANES)] = (w16 >> 16) | (a16 & mask)
        return carry
      lax.fori_loop(0, chunk, pack_row, 0)
      dst = pl.multiple_of(base + c * chunk, chunk)
      pltpu.sync_copy(obuf, out_hbm.at[pl.ds(dst, chunk)])

    gather(0, wbuf0, abuf0, sem_w0, sem_a0)

    def body(i, carry):
      c0 = 2 * i
      gather(c0 + 1, wbuf1, abuf1, sem_w1, sem_a1)
      wait(wbuf0, abuf0, sem_w0, sem_a0)
      pack_store(c0, wbuf0, abuf0)

      @pl.when(c0 + 2 < n_chunks)
      def _():
        gather(c0 + 2, wbuf0, abuf0, sem_w0, sem_a0)

      wait(wbuf1, abuf1, sem_w1, sem_a1)
      pack_store(c0 + 1, wbuf1, abuf1)
      return carry

    lax.fori_loop(0, n_pairs, body, 0)
    if tail:
      wait(wbuf0, abuf0, sem_w0, sem_a0)
      pack_store(n_chunks - 1, wbuf0, abuf0)

  return k(word_tab, aux_tab, widx, aidx)


def _fast_sin(x):
  y = x * INV_2PI
  k = (y + RND_MAGIC) - RND_MAGIC
  t = y - k
  u = t * t
  return t * (S0 + u * (S1 + u * (S2 + u * (S3 + u * S4))))


def _tc_math(prev, packed, deltas, ages, wc, wta, cwc, cphc,
             b, gamma, beta, BL, H, T, TB, off):
  nb = packed.shape[0] // TB

  def body(*refs):
    if prev is None:
      (p_ref, d_ref, ag_ref, wc_ref, wta_ref,
       cw_ref, cphi_ref, b_ref, gm_ref, bt_ref, out_ref) = refs
    else:
      (_, p_ref, d_ref, ag_ref, wc_ref, wta_ref,
       cw_ref, cphi_ref, b_ref, gm_ref, bt_ref, out_ref) = refs
    d = d_ref[0]
    a = ag_ref[0]
    cwv = cw_ref[...]
    cph = cphi_ref[...]
    ph = jnp.concatenate(
        [d * cwv[:T] + cph[:T],
         a * cwv[T:] + cph[T:]], axis=0)
    feats_t = _fast_sin(ph)
    p = p_ref[...]
    code = lax.bitcast_convert_type(p << 16, jnp.float32)
    aux_f = lax.bitcast_convert_type(p & jnp.uint32(MASK_HI), jnp.float32)
    acc = jnp.dot(code, wc_ref[...], preferred_element_type=jnp.float32)
    acc += lax.dot_general(feats_t, wta_ref[...], (((0,), (0,)), ((), ())),
                           preferred_element_type=jnp.float32)
    tok = jnp.tanh(acc + b_ref[...]) + aux_f
    mu = jnp.mean(tok, axis=1, keepdims=True)
    var = jnp.mean(jnp.square(tok - mu), axis=1, keepdims=True)
    out_ref[...] = ((tok - mu) * lax.rsqrt(var + 1e-12)
                    * gm_ref[...] + bt_ref[...])

  full = lambda r, c: pl.BlockSpec((r, c), lambda i: (0, 0))
  in_specs = [
      pl.BlockSpec((TB, H), lambda i: (i, 0)),
      pl.BlockSpec((1, 1, TB), lambda i: (i, 0, 0)),
      pl.BlockSpec((1, 1, TB), lambda i: (i, 0, 0)),
      full(H, H), full(2 * T, H),
      full(2 * T, 1), full(2 * T, 1),
      full(1, H), full(1, H), full(1, H),
  ]
  args = (packed, deltas, ages, wc, wta, cwc, cphc, b, gamma, beta)
  aliases = {}
  if prev is not None:
    in_specs = [pl.BlockSpec(memory_space=pltpu.MemorySpace.HBM)] + in_specs
    args = (prev,) + args
    aliases = {0: 0}
  return pl.pallas_call(
      body,
      grid=(nb,),
      in_specs=in_specs,
      out_specs=pl.BlockSpec((TB, H), lambda i: (i + off, 0)),
      out_shape=jax.ShapeDtypeStruct((BL, H), jnp.float32),
      input_output_aliases=aliases,
      compiler_params=pltpu.CompilerParams(
          dimension_semantics=("arbitrary",)),
  )(*args)


def kernel(input_ids, token_type_ids, time_stamps, ages, visit_orders,
           visit_segments, word_emb, type_emb, order_emb, seg_emb,
           time_w, time_phi, age_w, age_phi, proj_W, proj_b,
           ln_gamma, ln_beta):
  B, Lx = input_ids.shape
  V, H = word_emb.shape
  T = time_w.shape[1]
  n_type, n_seg, n_order = type_emb.shape[0], seg_emb.shape[0], order_emb.shape[0]
  BL = B * Lx

  aux_tab = ((type_emb[:, None, :] + seg_emb[None, :, :])
             .reshape(n_type * n_seg, H)[:, None, :]
             + order_emb[None, :, :]).reshape(n_type * n_seg * n_order, H)
  aidx = ((token_type_ids * n_seg + visit_segments) * n_order
          + visit_orders).reshape(BL).astype(jnp.int32)
  widx = input_ids.reshape(BL).astype(jnp.int32)

  word_u = lax.bitcast_convert_type(word_emb, jnp.uint32)
  aux_u = lax.bitcast_convert_type(aux_tab, jnp.uint32)

  TB = 2048
  NSLICE = 4
  SL = BL // NSLICE
  deltas = jnp.concatenate(
      [time_stamps[:, :1] * 0.0, time_stamps[:, 1:] - time_stamps[:, :-1]],
      axis=-1).reshape(BL // TB, 1, TB)
  ages2 = ages.reshape(BL // TB, 1, TB)
  wc = proj_W[:H]
  wta = proj_W[H:]
  cwc = jnp.concatenate([time_w, age_w], axis=1).reshape(2 * T, 1)
  cphc = jnp.concatenate([time_phi, age_phi], axis=1).reshape(2 * T, 1)
  bb = proj_b.reshape(1, H)
  gm = ln_gamma.reshape(1, H)
  bt = ln_beta.reshape(1, H)

  nbs = SL // TB
  packed = [
      _sc_gather_pack(word_u, aux_u,
                      widx[s * SL:(s + 1) * SL], aidx[s * SL:(s + 1) * SL],
                      SL, H)
      for s in range(NSLICE)
  ]
  out = None
  for s in range(NSLICE):
    out = _tc_math(out, packed[s],
                   deltas[s * nbs:(s + 1) * nbs], ages2[s * nbs:(s + 1) * nbs],
                   wc, wta, cwc, cphc, bb, gm, bt,
                   BL, H, T, TB, off=s * nbs)
  return out.reshape(B, Lx, H)


# speedup vs baseline: 1.9634x; 1.0718x over previous
import functools

import jax
import jax.numpy as jnp
from jax import lax
from jax.experimental import pallas as pl
from jax.experimental.pallas import tpu as pltpu
from jax.experimental.pallas import tpu_sc as plsc

NC, NS = 2, 16
NW = NC * NS
CHUNK = 128
LANES = 16
MASK_HI = 0xFFFF0000

INV_2PI = 0.15915493667125702
RND_MAGIC = 12582912.0
S0 = 6.283055994859666
S1 = -41.331226406885634
S2 = 81.36701207816412
S3 = -74.47917011197654
S4 = 32.78367310635748


def _sc_gather_pack(word_tab, aux_tab, widx, aidx, BL, H):
  per_w = BL // NW
  chunk = next(c for c in (CHUNK, 80, 64, 40, 32, 16, 8) if per_w % c == 0)
  n_chunks = per_w // chunk
  n_pairs = n_chunks // 2
  tail = n_chunks % 2
  mesh = plsc.VectorSubcoreMesh(core_axis_name="c", subcore_axis_name="s",
                                num_cores=NC, num_subcores=NS)

  n_order = aux_tab.shape[0]

  @functools.partial(
      pl.kernel,
      out_type=jax.ShapeDtypeStruct((BL, H), jnp.uint32),
      mesh=mesh,
      scratch_types=[
          pltpu.VMEM((per_w,), jnp.int32),
          pltpu.VMEM((per_w,), jnp.int32),
          pltpu.VMEM((chunk, H), jnp.uint32),
          pltpu.VMEM((chunk, H), jnp.uint32),
          pltpu.VMEM((chunk, H), jnp.uint32),
          pltpu.VMEM((chunk, H), jnp.uint32),
          pltpu.VMEM((chunk, H), jnp.uint32),
          pltpu.VMEM_SHARED((n_order, H), jnp.uint32),
          pltpu.SemaphoreType.DMA,
          pltpu.SemaphoreType.DMA,
          pltpu.SemaphoreType.DMA,
          pltpu.SemaphoreType.DMA,
      ],
  )
  def k(word_hbm, aux_hbm, widx_hbm, aidx_hbm, out_hbm,
        widx_v, aidx_v, wbuf0, abuf0, wbuf1, abuf1, obuf, order_sh,
        sem_w0, sem_a0, sem_w1, sem_a1):
    wid = lax.axis_index("s") * NC + lax.axis_index("c")
    base = pl.multiple_of(wid * per_w, per_w)

    @pl.when(lax.axis_index("s") == 0)
    def _():
      pltpu.sync_copy(aux_hbm, order_sh)
    pltpu.sync_copy(widx_hbm.at[pl.ds(base, per_w)], widx_v)
    pltpu.sync_copy(aidx_hbm.at[pl.ds(base, per_w)], aidx_v)
    plsc.subcore_barrier()

    def gather(c, wbuf, abuf, sem_w, sem_a):
      off = pl.multiple_of(c * chunk, chunk)
      pltpu.async_copy(word_hbm.at[widx_v.at[pl.ds(off, chunk)]], wbuf, sem_w)
      pltpu.async_copy(order_sh.at[aidx_v.at[pl.ds(off, chunk)]], abuf, sem_a)

    def wait(wbuf, abuf, sem_w, sem_a):
      pltpu.make_async_copy(word_hbm.at[widx_v.at[pl.ds(0, chunk)]],
                            wbuf, sem_w).wait()
      pltpu.make_async_copy(order_sh.at[aidx_v.at[pl.ds(0, chunk)]],
                            abuf, sem_a).wait()

    def pack_store(c, wbuf, abuf):
      mask = jnp.uint32(MASK_HI)
      def pack_row(t, carry):
        for kk in range(H // LANES):
          w16 = wbuf[t, pl.ds(LANES * kk, LANES)]
          a16 = abuf[t, pl.ds(LANES * kk, LANES)]
          obuf[t, pl.ds(LANES * kk, LANES)] = (w16 >> 16) | (a16 & mask)
        return carry
      lax.fori_loop(0, chunk, pack_row, 0)
      dst = pl.multiple_of(base + c * chunk, chunk)
      pltpu.sync_copy(obuf, out_hbm.at[pl.ds(dst, chunk)])

    gather(0, wbuf0, abuf0, sem_w0, sem_a0)

    def body(i, carry):
      c0 = 2 * i
      gather(c0 + 1, wbuf1, abuf1, sem_w1, sem_a1)
      wait(wbuf0, abuf0, sem_w0, sem_a0)
      pack_store(c0, wbuf0, abuf0)

      @pl.when(c0 + 2 < n_chunks)
      def _():
        gather(c0 + 2, wbuf0, abuf0, sem_w0, sem_a0)

      wait(wbuf1, abuf1, sem_w1, sem_a1)
      pack_store(c0 + 1, wbuf1, abuf1)
      return carry

    lax.fori_loop(0, n_pairs, body, 0)
    if tail:
      wait(wbuf0, abuf0, sem_w0, sem_a0)
      pack_store(n_chunks - 1, wbuf0, abuf0)

  return k(word_tab, aux_tab, widx, aidx)


def _fast_sin(x):
  y = x * INV_2PI
  k = (y + RND_MAGIC) - RND_MAGIC
  t = y - k
  u = t * t
  return t * (S0 + u * (S1 + u * (S2 + u * (S3 + u * S4))))


def _tc_math(prev, packed, deltas, ages, tsidx, tstab, wc, wta, cwc, cphc,
             b, gamma, beta, BL, H, T, TB, off):
  nb = packed.shape[0] // TB

  def body(*refs):
    if prev is None:
      (p_ref, d_ref, ag_ref, ts_ref, tstab_ref, wc_ref, wta_ref,
       cw_ref, cphi_ref, b_ref, gm_ref, bt_ref, out_ref) = refs
    else:
      (_, p_ref, d_ref, ag_ref, ts_ref, tstab_ref, wc_ref, wta_ref,
       cw_ref, cphi_ref, b_ref, gm_ref, bt_ref, out_ref) = refs
    d = d_ref[0]
    a = ag_ref[0]
    cwv = cw_ref[...]
    cph = cphi_ref[...]
    ph = jnp.concatenate(
        [d * cwv[:T] + cph[:T],
         a * cwv[T:] + cph[T:]], axis=0)
    feats_t = _fast_sin(ph)
    p = p_ref[...]
    code = lax.bitcast_convert_type(p << 16, jnp.float32)
    aux_f = lax.bitcast_convert_type(p & jnp.uint32(MASK_HI), jnp.float32)
    acc = jnp.dot(code, wc_ref[...], preferred_element_type=jnp.float32)
    acc += lax.dot_general(feats_t, wta_ref[...], (((0,), (0,)), ((), ())),
                           preferred_element_type=jnp.float32)
    ts = ts_ref[0]
    nts = tstab_ref.shape[0]
    rows = lax.broadcasted_iota(jnp.int32, (nts, TB), 0)
    onehot = jnp.where(rows == ts, 1.0, 0.0)
    ts_add = lax.dot_general(onehot, tstab_ref[...], (((0,), (0,)), ((), ())),
                             preferred_element_type=jnp.float32)
    tok = jnp.tanh(acc + b_ref[...]) + aux_f + ts_add
    mu = jnp.mean(tok, axis=1, keepdims=True)
    var = jnp.mean(jnp.square(tok - mu), axis=1, keepdims=True)
    out_ref[...] = ((tok - mu) * lax.rsqrt(var + 1e-12)
                    * gm_ref[...] + bt_ref[...])

  full = lambda r, c: pl.BlockSpec((r, c), lambda i: (0, 0))
  in_specs = [
      pl.BlockSpec((TB, H), lambda i: (i, 0)),
      pl.BlockSpec((1, 1, TB), lambda i: (i, 0, 0)),
      pl.BlockSpec((1, 1, TB), lambda i: (i, 0, 0)),
      pl.BlockSpec((1, 1, TB), lambda i: (i, 0, 0)),
      full(tstab.shape[0], H),
      full(H, H), full(2 * T, H),
      full(2 * T, 1), full(2 * T, 1),
      full(1, H), full(1, H), full(1, H),
  ]
  args = (packed, deltas, ages, tsidx, tstab, wc, wta, cwc, cphc,
          b, gamma, beta)
  aliases = {}
  if prev is not None:
    in_specs = [pl.BlockSpec(memory_space=pltpu.MemorySpace.HBM)] + in_specs
    args = (prev,) + args
    aliases = {0: 0}
  return pl.pallas_call(
      body,
      grid=(nb,),
      in_specs=in_specs,
      out_specs=pl.BlockSpec((TB, H), lambda i: (i + off, 0)),
      out_shape=jax.ShapeDtypeStruct((BL, H), jnp.float32),
      input_output_aliases=aliases,
      compiler_params=pltpu.CompilerParams(
          dimension_semantics=("arbitrary",)),
  )(*args)


def kernel(input_ids, token_type_ids, time_stamps, ages, visit_orders,
           visit_segments, word_emb, type_emb, order_emb, seg_emb,
           time_w, time_phi, age_w, age_phi, proj_W, proj_b,
           ln_gamma, ln_beta):
  B, Lx = input_ids.shape
  V, H = word_emb.shape
  T = time_w.shape[1]
  n_type, n_seg, n_order = type_emb.shape[0], seg_emb.shape[0], order_emb.shape[0]
  BL = B * Lx

  tstab = jnp.concatenate(
      [(type_emb[:, None, :] + seg_emb[None, :, :]).reshape(n_type * n_seg, H),
       jnp.zeros((32 - n_type * n_seg, H), jnp.float32)], axis=0)
  aidx = visit_orders.reshape(BL).astype(jnp.int32)
  widx = input_ids.reshape(BL).astype(jnp.int32)

  word_u = lax.bitcast_convert_type(word_emb, jnp.uint32)
  aux_u = lax.bitcast_convert_type(order_emb, jnp.uint32)

  TB = 2048
  NSLICE = 4
  SL = BL // NSLICE
  deltas = jnp.concatenate(
      [time_stamps[:, :1] * 0.0, time_stamps[:, 1:] - time_stamps[:, :-1]],
      axis=-1).reshape(BL // TB, 1, TB)
  ages2 = ages.reshape(BL // TB, 1, TB)
  tsidx = (token_type_ids * n_seg + visit_segments).reshape(
      BL // TB, 1, TB).astype(jnp.int32)
  wc = proj_W[:H]
  wta = proj_W[H:]
  cwc = jnp.concatenate([time_w, age_w], axis=1).reshape(2 * T, 1)
  cphc = jnp.concatenate([time_phi, age_phi], axis=1).reshape(2 * T, 1)
  bb = proj_b.reshape(1, H)
  gm = ln_gamma.reshape(1, H)
  bt = ln_beta.reshape(1, H)

  nbs = SL // TB
  packed = [
      _sc_gather_pack(word_u, aux_u,
                      widx[s * SL:(s + 1) * SL], aidx[s * SL:(s + 1) * SL],
                      SL, H)
      for s in range(NSLICE)
  ]
  out = None
  for s in range(NSLICE):
    out = _tc_math(out, packed[s],
                   deltas[s * nbs:(s + 1) * nbs], ages2[s * nbs:(s + 1) * nbs],
                   tsidx[s * nbs:(s + 1) * nbs], tstab,
                   wc, wta, cwc, cphc, bb, gm, bt,
                   BL, H, T, TB, off=s * nbs)
  return out.reshape(B, Lx, H)
